# baseline (speedup 1.0000x reference)
import functools

import jax
import jax.numpy as jnp
from jax import lax
from jax.experimental import pallas as pl
from jax.experimental.pallas import tpu as pltpu
from jax.experimental.pallas import tpu_sc as plsc

D_MODEL = 768
N_EXPERTS = 8
TOP_K = 2
D_FF = 768
T_TOKENS = 2048
N_PAIRS = T_TOKENS * TOP_K
BLK = 128
N_BLOCKS = (N_PAIRS + N_EXPERTS * (BLK - 1) + BLK - 1) // BLK
PAD_N = N_BLOCKS * BLK
C_CHUNK = 128


def _route_body(x_ref, gate_w_ref, slots_ref, wpair_ref, b2e_ref,
                occ_ref, ranks_ref):
    x = x_ref[...]
    logits = jnp.dot(x, gate_w_ref[...], preferred_element_type=jnp.float32)
    z = logits - jnp.max(logits, axis=1, keepdims=True)
    ez = jnp.exp(z)
    p = ez / jnp.sum(ez, axis=1, keepdims=True)

    lane = lax.broadcasted_iota(jnp.int32, (T_TOKENS, N_EXPERTS), 1)
    m1 = jnp.max(p, axis=1, keepdims=True)
    i1 = jnp.min(jnp.where(p == m1, lane, N_EXPERTS), axis=1, keepdims=True)
    sel1 = lane == i1
    p2 = jnp.where(sel1, -1.0, p)
    m2 = jnp.max(p2, axis=1, keepdims=True)
    i2 = jnp.min(jnp.where(p2 == m2, lane, N_EXPERTS), axis=1, keepdims=True)
    sel2 = lane == i2
    s = m1 + m2

    occ_ref[...] = jnp.where(sel1 | sel2, 1.0, 0.0)

    r = lax.broadcasted_iota(jnp.int32, (C_CHUNK, C_CHUNK), 0)
    c = lax.broadcasted_iota(jnp.int32, (C_CHUNK, C_CHUNK), 1)
    tril = (r > c).astype(jnp.bfloat16)

    def chunk(i, offset):
        blk = occ_ref[pl.ds(i * C_CHUNK, C_CHUNK), :]
        ranks_ref[pl.ds(i * C_CHUNK, C_CHUNK), :] = (
            jnp.dot(tril, blk.astype(jnp.bfloat16),
                    preferred_element_type=jnp.float32) + offset)
        return offset + jnp.sum(blk, axis=0, keepdims=True)

    counts = lax.fori_loop(0, T_TOKENS // C_CHUNK, chunk,
                           jnp.zeros((1, N_EXPERTS), jnp.float32))

    cb = jnp.ceil(counts * (1.0 / BLK)) * float(BLK)
    r8 = lax.broadcasted_iota(jnp.int32, (N_EXPERTS, N_EXPERTS), 0)
    c8 = lax.broadcasted_iota(jnp.int32, (N_EXPERTS, N_EXPERTS), 1)
    upper = (r8 < c8).astype(jnp.float32)
    base = jnp.dot(cb, upper, preferred_element_type=jnp.float32)

    eye8 = (r8 == c8).astype(jnp.float32)
    bb_col = lax.dot_general(eye8, base * (1.0 / BLK),
                             (((1,), (1,)), ((), ())),
                             preferred_element_type=jnp.float32)
    blocks = lax.broadcasted_iota(jnp.int32, (1, N_BLOCKS), 1).astype(jnp.float32)
    b2e = jnp.sum((bb_col <= blocks).astype(jnp.int32), axis=0,
                  keepdims=True) - 1
    b2e_ref[...] = b2e

    ranks = ranks_ref[...]
    rank1 = jnp.sum(jnp.where(sel1, ranks, 0.0), axis=1, keepdims=True)
    rank2 = jnp.sum(jnp.where(sel2, ranks, 0.0), axis=1, keepdims=True)
    base1 = jnp.sum(jnp.where(sel1, base, 0.0), axis=1, keepdims=True)
    base2 = jnp.sum(jnp.where(sel2, base, 0.0), axis=1, keepdims=True)
    slot1 = (base1 + rank1).astype(jnp.int32)
    slot2 = (base2 + rank2).astype(jnp.int32)
    slots_ref[...] = jnp.concatenate([slot1, slot2], axis=1)
    wpair_ref[...] = jnp.concatenate([m1 / s, m2 / s], axis=1)


def _route(x, gate_w):
    return pl.pallas_call(
        _route_body,
        in_specs=[
            pl.BlockSpec((T_TOKENS, D_MODEL), lambda: (0, 0)),
            pl.BlockSpec((D_MODEL, N_EXPERTS), lambda: (0, 0)),
        ],
        out_specs=[
            pl.BlockSpec((T_TOKENS, TOP_K), lambda: (0, 0)),
            pl.BlockSpec((T_TOKENS, TOP_K), lambda: (0, 0)),
            pl.BlockSpec((1, N_BLOCKS), lambda: (0, 0)),
        ],
        out_shape=[
            jax.ShapeDtypeStruct((T_TOKENS, TOP_K), jnp.int32),
            jax.ShapeDtypeStruct((T_TOKENS, TOP_K), jnp.float32),
            jax.ShapeDtypeStruct((1, N_BLOCKS), jnp.int32),
        ],
        scratch_shapes=[
            pltpu.VMEM((T_TOKENS, N_EXPERTS), jnp.float32),
            pltpu.VMEM((T_TOKENS, N_EXPERTS), jnp.float32),
        ],
    )(x, gate_w)


def _make_sc_gather():
    info = plsc.get_sparse_core_info()
    NC, NS = info.num_cores, info.num_subcores
    NW = NC * NS
    stripe = PAD_N // NW
    gchunk = stripe // 2
    ppw = N_PAIRS // NS
    GSUB = 8
    zchunk = PAD_N // NS
    mesh = plsc.VectorSubcoreMesh(core_axis_name="c", subcore_axis_name="s")

    @functools.partial(
        pl.kernel, mesh=mesh,
        out_type=[
            jax.ShapeDtypeStruct((PAD_N, D_MODEL), jnp.float32),
            jax.ShapeDtypeStruct((PAD_N,), jnp.float32),
        ],
        scratch_types=[
            pltpu.VMEM((2, 128), jnp.int32),
            pltpu.VMEM((2, 128), jnp.int32),
            pltpu.VMEM((2, 128), jnp.float32),
            pltpu.VMEM((zchunk,), jnp.int32),
            pltpu.VMEM((zchunk,), jnp.float32),
            pltpu.VMEM((stripe,), jnp.int32),
            pltpu.VMEM((stripe,), jnp.float32),
            pltpu.VMEM((2, gchunk, D_MODEL), jnp.float32),
            pltpu.VMEM_SHARED((PAD_N,), jnp.int32),
            pltpu.VMEM_SHARED((PAD_N,), jnp.float32),
            pltpu.SemaphoreType.DMA,
            pltpu.SemaphoreType.DMA,
        ],
        compiler_params=pltpu.CompilerParams(needs_layout_passes=False),
    )
    def sc_gather(x_hbm, slots_hbm, w_hbm, xs_hbm, ws_hbm,
                  idx_v, tok_v, wv_v, zero_v, zerof_v, tstr_v, wstr_v, rows_v,
                  tok_sh, wm_sh, sem0, sem1):
        cid = lax.axis_index("c")
        sid = lax.axis_index("s")
        wid = sid * NC + cid
        pbase = sid * ppw

        iota16 = lax.iota(jnp.int32, 16)
        for h in range(2):
            pltpu.sync_copy(slots_hbm.at[pl.ds(pbase + h * 128, 128)],
                            idx_v.at[h])
            pltpu.sync_copy(w_hbm.at[pl.ds(pbase + h * 128, 128)],
                            wv_v.at[h])
            for i in range(8):
                tok_v[h, pl.ds(i * 16, 16)] = (
                    iota16 + (pbase + h * 128 + i * 16)) >> 1

        z16 = jnp.zeros((16,), jnp.int32)
        z16f = jnp.zeros((16,), jnp.float32)
        for i in range(zchunk // 16):
            zero_v[pl.ds(i * 16, 16)] = z16
            zerof_v[pl.ds(i * 16, 16)] = z16f
        pltpu.sync_copy(zero_v, tok_sh.at[pl.ds(sid * zchunk, zchunk)])
        pltpu.sync_copy(zerof_v, wm_sh.at[pl.ds(sid * zchunk, zchunk)])
        plsc.subcore_barrier()

        for h in range(2):
            pltpu.sync_copy(tok_v.at[h], tok_sh.at[idx_v.at[h]])
            pltpu.sync_copy(wv_v.at[h], wm_sh.at[idx_v.at[h]])
        plsc.subcore_barrier()

        base = wid * stripe
        pltpu.sync_copy(tok_sh.at[pl.ds(base, stripe)], tstr_v)
        pltpu.sync_copy(wm_sh.at[pl.ds(base, stripe)], wstr_v)
        pltpu.sync_copy(wstr_v, ws_hbm.at[pl.ds(base, stripe)])
        sems = (sem0, sem1)
        nfire = gchunk // GSUB
        cps = [[], []]
        for h in range(2):
            for f in range(nfire):
                o = h * gchunk + f * GSUB
                idx_ref = tstr_v.at[pl.ds(o, GSUB)]
                dst = rows_v.at[h, pl.ds(f * GSUB, GSUB)]
                cps[h].append(pltpu.async_copy(x_hbm.at[idx_ref], dst,
                                               sems[h]))
        for h in range(2):
            for cp in cps[h]:
                cp.wait()
            pltpu.sync_copy(rows_v.at[h],
                            xs_hbm.at[pl.ds(base + h * gchunk, gchunk)])

    return sc_gather


def _gemm_body(b2e_ref, xs_ref, gup_ref, down_ref, w_ref, y_ref):
    xs = xs_ref[...].astype(jnp.bfloat16)
    gu = jnp.dot(xs, gup_ref[0].astype(jnp.bfloat16),
                 preferred_element_type=jnp.float32)
    g = gu[:, :D_FF]
    u = gu[:, D_FF:]
    act = (g * jax.nn.sigmoid(g) * u).astype(jnp.bfloat16)
    y = jnp.dot(act, down_ref[0].astype(jnp.bfloat16),
                preferred_element_type=jnp.float32)
    r = lax.broadcasted_iota(jnp.int32, (BLK, BLK), 0)
    c = lax.broadcasted_iota(jnp.int32, (BLK, BLK), 1)
    eye = (r == c).astype(jnp.float32)
    wcol = lax.dot_general(eye, w_ref[0], (((1,), (1,)), ((), ())),
                           preferred_element_type=jnp.float32)
    y_ref[...] = y * wcol


def _gemm(b2e, xs, gup, down, ws):
    grid_spec = pltpu.PrefetchScalarGridSpec(
        num_scalar_prefetch=1,
        grid=(N_BLOCKS,),
        in_specs=[
            pl.BlockSpec((BLK, D_MODEL), lambda b, b2e: (b, 0)),
            pl.BlockSpec((1, D_MODEL, 2 * D_FF),
                         lambda b, b2e: (b2e[0, b], 0, 0)),
            pl.BlockSpec((1, D_FF, D_MODEL),
                         lambda b, b2e: (b2e[0, b], 0, 0)),
            pl.BlockSpec((1, 1, BLK), lambda b, b2e: (b, 0, 0)),
        ],
        out_specs=pl.BlockSpec((BLK, D_MODEL), lambda b, b2e: (b, 0)),
    )
    return pl.pallas_call(
        _gemm_body,
        grid_spec=grid_spec,
        out_shape=jax.ShapeDtypeStruct((PAD_N, D_MODEL), jnp.float32),
    )(b2e, xs, gup, down, ws.reshape(N_BLOCKS, 1, BLK))


def _make_sc_combine():
    info = plsc.get_sparse_core_info()
    NC, NS = info.num_cores, info.num_subcores
    NW = NC * NS
    tpw = T_TOKENS // NW
    half = tpw // 2
    mesh = plsc.VectorSubcoreMesh(core_axis_name="c", subcore_axis_name="s")

    nchunk = 4
    tpc = tpw // nchunk
    ppc = 2 * tpc

    @functools.partial(
        pl.kernel, mesh=mesh,
        out_type=jax.ShapeDtypeStruct((T_TOKENS, D_MODEL), jnp.float32),
        scratch_types=[
            pltpu.VMEM((nchunk, ppc), jnp.int32),
            pltpu.VMEM((2, ppc, D_MODEL), jnp.float32),
            pltpu.VMEM((tpc, D_MODEL), jnp.float32),
            pltpu.SemaphoreType.DMA,
            pltpu.SemaphoreType.DMA,
        ],
        compiler_params=pltpu.CompilerParams(needs_layout_passes=False),
    )
    def sc_combine(y_hbm, slots_hbm, out_hbm, idx_v, rows_v, out_v,
                   sem0, sem1):
        wid = lax.axis_index("s") * NC + lax.axis_index("c")
        for c in range(nchunk):
            pltpu.sync_copy(
                slots_hbm.at[pl.ds(wid * 2 * tpw + c * ppc, ppc)],
                idx_v.at[c])
        nvec = D_MODEL // 16
        sems = (sem0, sem1)
        gsub = 8

        def fire(c, buf):
            h = []
            for f in range(ppc // gsub):
                idx_ref = idx_v.at[c, pl.ds(f * gsub, gsub)]
                dst = rows_v.at[buf, pl.ds(f * gsub, gsub)]
                h.append(pltpu.async_copy(y_hbm.at[idx_ref], dst, sems[buf]))
            return h

        cps = [None, None]
        cps[0] = fire(0, 0)
        for c in range(nchunk):
            buf = c % 2
            if c + 1 < nchunk:
                cps[1 - buf] = fire(c + 1, 1 - buf)
            for cp in cps[buf]:
                cp.wait()

            def tok(j, _):
                for v in range(nvec):
                    a = rows_v[buf, 2 * j, pl.ds(v * 16, 16)]
                    b = rows_v[buf, 2 * j + 1, pl.ds(v * 16, 16)]
                    out_v[j, pl.ds(v * 16, 16)] = a + b
                return 0

            lax.fori_loop(0, tpc, tok, 0)
            pltpu.sync_copy(out_v,
                            out_hbm.at[pl.ds(wid * tpw + c * tpc, tpc)])

    return sc_combine


def kernel(hidden_states, gate_w, gate_up_proj, down_proj):
    batch, seq, d = hidden_states.shape
    x = hidden_states.reshape(batch * seq, d)
    slots2, wpair, b2e = _route(x, gate_w)
    slots_flat = slots2.reshape(N_PAIRS)
    w_flat = wpair.reshape(N_PAIRS)
    xs, ws = _make_sc_gather()(x, slots_flat, w_flat)
    y = _gemm(b2e, xs, gate_up_proj, down_proj, ws)
    out = _make_sc_combine()(y, slots_flat)
    return out.reshape(batch, seq, d)


# speedup vs baseline: 1.4878x; 1.4878x over previous
import functools

import jax
import jax.numpy as jnp
from jax import lax
from jax.experimental import pallas as pl
from jax.experimental.pallas import tpu as pltpu
from jax.experimental.pallas import tpu_sc as plsc

D_MODEL = 768
N_EXPERTS = 8
TOP_K = 2
D_FF = 768
T_TOKENS = 2048
N_PAIRS = T_TOKENS * TOP_K
BLK = 128
N_BLOCKS = (N_PAIRS + N_EXPERTS * (BLK - 1) + BLK - 1) // BLK
PAD_N = N_BLOCKS * BLK
C_CHUNK = 128


def _route_body(x_ref, gate_w_ref, slots_ref, wpair_ref, b2e_ref,
                occ_ref, ranks_ref):
    x = x_ref[...]
    logits = jnp.dot(x, gate_w_ref[...], preferred_element_type=jnp.float32)
    z = logits - jnp.max(logits, axis=1, keepdims=True)
    ez = jnp.exp(z)
    p = ez / jnp.sum(ez, axis=1, keepdims=True)

    lane = lax.broadcasted_iota(jnp.int32, (T_TOKENS, N_EXPERTS), 1)
    m1 = jnp.max(p, axis=1, keepdims=True)
    i1 = jnp.min(jnp.where(p == m1, lane, N_EXPERTS), axis=1, keepdims=True)
    sel1 = lane == i1
    p2 = jnp.where(sel1, -1.0, p)
    m2 = jnp.max(p2, axis=1, keepdims=True)
    i2 = jnp.min(jnp.where(p2 == m2, lane, N_EXPERTS), axis=1, keepdims=True)
    sel2 = lane == i2
    s = m1 + m2

    occ_ref[...] = jnp.where(sel1 | sel2, 1.0, 0.0)

    r = lax.broadcasted_iota(jnp.int32, (C_CHUNK, C_CHUNK), 0)
    c = lax.broadcasted_iota(jnp.int32, (C_CHUNK, C_CHUNK), 1)
    tril = (r > c).astype(jnp.bfloat16)

    def chunk(i, offset):
        blk = occ_ref[pl.ds(i * C_CHUNK, C_CHUNK), :]
        ranks_ref[pl.ds(i * C_CHUNK, C_CHUNK), :] = (
            jnp.dot(tril, blk.astype(jnp.bfloat16),
                    preferred_element_type=jnp.float32) + offset)
        return offset + jnp.sum(blk, axis=0, keepdims=True)

    counts = lax.fori_loop(0, T_TOKENS // C_CHUNK, chunk,
                           jnp.zeros((1, N_EXPERTS), jnp.float32))

    cb = jnp.ceil(counts * (1.0 / BLK)) * float(BLK)
    r8 = lax.broadcasted_iota(jnp.int32, (N_EXPERTS, N_EXPERTS), 0)
    c8 = lax.broadcasted_iota(jnp.int32, (N_EXPERTS, N_EXPERTS), 1)
    upper = (r8 < c8).astype(jnp.float32)
    base = jnp.dot(cb, upper, preferred_element_type=jnp.float32)

    eye8 = (r8 == c8).astype(jnp.float32)
    bb_col = lax.dot_general(eye8, base * (1.0 / BLK),
                             (((1,), (1,)), ((), ())),
                             preferred_element_type=jnp.float32)
    blocks = lax.broadcasted_iota(jnp.int32, (1, N_BLOCKS), 1).astype(jnp.float32)
    b2e = jnp.sum((bb_col <= blocks).astype(jnp.int32), axis=0,
                  keepdims=True) - 1
    b2e_ref[...] = b2e

    ranks = ranks_ref[...]
    rank1 = jnp.sum(jnp.where(sel1, ranks, 0.0), axis=1, keepdims=True)
    rank2 = jnp.sum(jnp.where(sel2, ranks, 0.0), axis=1, keepdims=True)
    base1 = jnp.sum(jnp.where(sel1, base, 0.0), axis=1, keepdims=True)
    base2 = jnp.sum(jnp.where(sel2, base, 0.0), axis=1, keepdims=True)
    slot1 = (base1 + rank1).astype(jnp.int32)
    slot2 = (base2 + rank2).astype(jnp.int32)
    slots_ref[...] = jnp.concatenate([slot1, slot2], axis=1)
    wpair_ref[...] = jnp.concatenate([m1 / s, m2 / s], axis=1)


def _route(x, gate_w):
    return pl.pallas_call(
        _route_body,
        in_specs=[
            pl.BlockSpec((T_TOKENS, D_MODEL), lambda: (0, 0)),
            pl.BlockSpec((D_MODEL, N_EXPERTS), lambda: (0, 0)),
        ],
        out_specs=[
            pl.BlockSpec((T_TOKENS, TOP_K), lambda: (0, 0)),
            pl.BlockSpec((T_TOKENS, TOP_K), lambda: (0, 0)),
            pl.BlockSpec((1, N_BLOCKS), lambda: (0, 0)),
        ],
        out_shape=[
            jax.ShapeDtypeStruct((T_TOKENS, TOP_K), jnp.int32),
            jax.ShapeDtypeStruct((T_TOKENS, TOP_K), jnp.float32),
            jax.ShapeDtypeStruct((1, N_BLOCKS), jnp.int32),
        ],
        scratch_shapes=[
            pltpu.VMEM((T_TOKENS, N_EXPERTS), jnp.float32),
            pltpu.VMEM((T_TOKENS, N_EXPERTS), jnp.float32),
        ],
    )(x, gate_w)


def _make_sc_gather():
    info = plsc.get_sparse_core_info()
    NC, NS = info.num_cores, info.num_subcores
    NW = NC * NS
    stripe = PAD_N // NW
    gchunk = stripe // 2
    ppw = N_PAIRS // NS
    GSUB = 8
    zchunk = PAD_N // NS
    mesh = plsc.VectorSubcoreMesh(core_axis_name="c", subcore_axis_name="s")

    tpw = T_TOKENS // NW
    ppg = 2 * tpw

    @functools.partial(
        pl.kernel, mesh=mesh,
        out_type=[
            jax.ShapeDtypeStruct((PAD_N, D_MODEL), jnp.float32),
            jax.ShapeDtypeStruct((PAD_N,), jnp.float32),
        ],
        scratch_types=[
            pltpu.VMEM((2, 128), jnp.int32),
            pltpu.VMEM((2, 128), jnp.float32),
            pltpu.VMEM((zchunk,), jnp.float32),
            pltpu.VMEM((stripe,), jnp.float32),
            pltpu.VMEM((ppg,), jnp.int32),
            pltpu.VMEM((tpw, D_MODEL), jnp.float32),
            pltpu.VMEM_SHARED((PAD_N,), jnp.float32),
            pltpu.SemaphoreType.DMA,
            pltpu.SemaphoreType.DMA,
        ],
        compiler_params=pltpu.CompilerParams(needs_layout_passes=False),
    )
    def sc_gather(x_hbm, slots_hbm, w_hbm, xs_hbm, ws_hbm,
                  idx_v, wv_v, zerof_v, wstr_v, slot_v, xrows_v,
                  wm_sh, semr, semw):
        cid = lax.axis_index("c")
        sid = lax.axis_index("s")
        wid = sid * NC + cid
        pbase_sc = sid * ppw
        gbase = wid * ppg

        rows_cp = pltpu.async_copy(x_hbm.at[pl.ds(wid * tpw, tpw)],
                                   xrows_v, semr)
        pltpu.sync_copy(slots_hbm.at[pl.ds(gbase, ppg)], slot_v)

        for h in range(2):
            pltpu.sync_copy(slots_hbm.at[pl.ds(pbase_sc + h * 128, 128)],
                            idx_v.at[h])
            pltpu.sync_copy(w_hbm.at[pl.ds(pbase_sc + h * 128, 128)],
                            wv_v.at[h])
        z16f = jnp.zeros((16,), jnp.float32)
        for i in range(zchunk // 16):
            zerof_v[pl.ds(i * 16, 16)] = z16f
        pltpu.sync_copy(zerof_v, wm_sh.at[pl.ds(sid * zchunk, zchunk)])
        plsc.subcore_barrier()
        for h in range(2):
            pltpu.sync_copy(wv_v.at[h], wm_sh.at[idx_v.at[h]])
        plsc.subcore_barrier()
        base = wid * stripe
        pltpu.sync_copy(wm_sh.at[pl.ds(base, stripe)], wstr_v)
        pltpu.sync_copy(wstr_v, ws_hbm.at[pl.ds(base, stripe)])

        rows_cp.wait()
        iota16 = lax.iota(jnp.int32, 16)

        def fire(p, _):
            chunk = slot_v[pl.ds((p >> 4) * 16, 16)]
            s = jnp.sum(jnp.where(iota16 == (p & 15), chunk, 0))
            pltpu.async_copy(xrows_v.at[pl.ds(p >> 1, 1)],
                             xs_hbm.at[pl.ds(s, 1)], semw)
            return 0

        lax.fori_loop(0, ppg, fire, 0)
        for h in range(2):
            pltpu.make_async_copy(x_hbm.at[pl.ds(0, tpw)], xrows_v,
                                  semw).wait()

    return sc_gather


def _gemm_body(b2e_ref, xs_ref, gup_ref, down_ref, w_ref, y_ref):
    xs = xs_ref[...].astype(jnp.bfloat16)
    gu = jnp.dot(xs, gup_ref[0].astype(jnp.bfloat16),
                 preferred_element_type=jnp.float32)
    g = gu[:, :D_FF]
    u = gu[:, D_FF:]
    act = (g * jax.nn.sigmoid(g) * u).astype(jnp.bfloat16)
    y = jnp.dot(act, down_ref[0].astype(jnp.bfloat16),
                preferred_element_type=jnp.float32)
    r = lax.broadcasted_iota(jnp.int32, (BLK, BLK), 0)
    c = lax.broadcasted_iota(jnp.int32, (BLK, BLK), 1)
    eye = (r == c).astype(jnp.float32)
    wcol = lax.dot_general(eye, w_ref[0], (((1,), (1,)), ((), ())),
                           preferred_element_type=jnp.float32)
    y_ref[...] = y * wcol


def _gemm(b2e, xs, gup, down, ws):
    grid_spec = pltpu.PrefetchScalarGridSpec(
        num_scalar_prefetch=1,
        grid=(N_BLOCKS,),
        in_specs=[
            pl.BlockSpec((BLK, D_MODEL), lambda b, b2e: (b, 0)),
            pl.BlockSpec((1, D_MODEL, 2 * D_FF),
                         lambda b, b2e: (b2e[0, b], 0, 0)),
            pl.BlockSpec((1, D_FF, D_MODEL),
                         lambda b, b2e: (b2e[0, b], 0, 0)),
            pl.BlockSpec((1, 1, BLK), lambda b, b2e: (b, 0, 0)),
        ],
        out_specs=pl.BlockSpec((BLK, D_MODEL), lambda b, b2e: (b, 0)),
    )
    return pl.pallas_call(
        _gemm_body,
        grid_spec=grid_spec,
        out_shape=jax.ShapeDtypeStruct((PAD_N, D_MODEL), jnp.float32),
    )(b2e, xs, gup, down, ws.reshape(N_BLOCKS, 1, BLK))


def _make_sc_combine():
    info = plsc.get_sparse_core_info()
    NC, NS = info.num_cores, info.num_subcores
    NW = NC * NS
    tpw = T_TOKENS // NW
    half = tpw // 2
    mesh = plsc.VectorSubcoreMesh(core_axis_name="c", subcore_axis_name="s")

    nchunk = 4
    tpc = tpw // nchunk
    ppc = 2 * tpc

    @functools.partial(
        pl.kernel, mesh=mesh,
        out_type=jax.ShapeDtypeStruct((T_TOKENS, D_MODEL), jnp.float32),
        scratch_types=[
            pltpu.VMEM((2 * tpw,), jnp.int32),
            pltpu.VMEM((2, ppc, D_MODEL), jnp.float32),
            pltpu.VMEM((tpc, D_MODEL), jnp.float32),
            pltpu.SemaphoreType.DMA,
            pltpu.SemaphoreType.DMA,
        ],
        compiler_params=pltpu.CompilerParams(needs_layout_passes=False),
    )
    def sc_combine(y_hbm, slots_hbm, out_hbm, idx_v, rows_v, out_v,
                   sem0, sem1):
        wid = lax.axis_index("s") * NC + lax.axis_index("c")
        pltpu.sync_copy(slots_hbm.at[pl.ds(wid * 2 * tpw, 2 * tpw)], idx_v)
        nvec = D_MODEL // 16
        sems = (sem0, sem1)
        iota16 = lax.iota(jnp.int32, 16)

        def fire(c, buf):
            def body(i, _):
                p = c * ppc + i
                chunk = idx_v[pl.ds((p >> 4) * 16, 16)]
                s = jnp.sum(jnp.where(iota16 == (p & 15), chunk, 0))
                pltpu.async_copy(y_hbm.at[pl.ds(s, 1)],
                                 rows_v.at[buf, pl.ds(i, 1)], sems[buf])
                return 0
            lax.fori_loop(0, ppc, body, 0)

        def drain(buf):
            pltpu.make_async_copy(y_hbm.at[pl.ds(0, ppc)],
                                  rows_v.at[buf], sems[buf]).wait()

        fire(0, 0)
        for c in range(nchunk):
            buf = c % 2
            if c + 1 < nchunk:
                fire(c + 1, 1 - buf)
            drain(buf)

            def tok(j, _):
                for v in range(nvec):
                    a = rows_v[buf, 2 * j, pl.ds(v * 16, 16)]
                    b = rows_v[buf, 2 * j + 1, pl.ds(v * 16, 16)]
                    out_v[j, pl.ds(v * 16, 16)] = a + b
                return 0

            lax.fori_loop(0, tpc, tok, 0)
            pltpu.sync_copy(out_v,
                            out_hbm.at[pl.ds(wid * tpw + c * tpc, tpc)])

    return sc_combine


def kernel(hidden_states, gate_w, gate_up_proj, down_proj):
    batch, seq, d = hidden_states.shape
    x = hidden_states.reshape(batch * seq, d)
    slots2, wpair, b2e = _route(x, gate_w)
    slots_flat = slots2.reshape(N_PAIRS)
    w_flat = wpair.reshape(N_PAIRS)
    xs, ws = _make_sc_gather()(x, slots_flat, w_flat)
    y = _gemm(b2e, xs, gate_up_proj, down_proj, ws)
    out = _make_sc_combine()(y, slots_flat)
    return out.reshape(batch, seq, d)


# speedup vs baseline: 1.5867x; 1.0664x over previous
import functools

import jax
import jax.numpy as jnp
from jax import lax
from jax.experimental import pallas as pl
from jax.experimental.pallas import tpu as pltpu
from jax.experimental.pallas import tpu_sc as plsc

D_MODEL = 768
N_EXPERTS = 8
TOP_K = 2
D_FF = 768
T_TOKENS = 2048
N_PAIRS = T_TOKENS * TOP_K
BLK = 128
N_BLOCKS = (N_PAIRS + N_EXPERTS * (BLK - 1) + BLK - 1) // BLK
PAD_N = N_BLOCKS * BLK
C_CHUNK = 128
D_HALF = D_MODEL // 2


def _route_body(x_ref, gate_w_ref, slots_ref, wpair_ref, b2e_ref, xbf_ref,
                occ_ref, ranks_ref):
    x = x_ref[...]
    xb = x.astype(jnp.bfloat16)
    lo32 = lax.bitcast_convert_type(xb[:, :D_HALF], jnp.int16).astype(jnp.int32)
    hi32 = lax.bitcast_convert_type(xb[:, D_HALF:], jnp.int16).astype(jnp.int32)
    xbf_ref[...] = (lo32 & 0xFFFF) | (hi32 << 16)
    logits = jnp.dot(x, gate_w_ref[...], preferred_element_type=jnp.float32)
    z = logits - jnp.max(logits, axis=1, keepdims=True)
    ez = jnp.exp(z)
    p = ez / jnp.sum(ez, axis=1, keepdims=True)

    lane = lax.broadcasted_iota(jnp.int32, (T_TOKENS, N_EXPERTS), 1)
    m1 = jnp.max(p, axis=1, keepdims=True)
    i1 = jnp.min(jnp.where(p == m1, lane, N_EXPERTS), axis=1, keepdims=True)
    sel1 = lane == i1
    p2 = jnp.where(sel1, -1.0, p)
    m2 = jnp.max(p2, axis=1, keepdims=True)
    i2 = jnp.min(jnp.where(p2 == m2, lane, N_EXPERTS), axis=1, keepdims=True)
    sel2 = lane == i2
    s = m1 + m2

    occ_ref[...] = jnp.where(sel1 | sel2, 1.0, 0.0)

    r = lax.broadcasted_iota(jnp.int32, (C_CHUNK, C_CHUNK), 0)
    c = lax.broadcasted_iota(jnp.int32, (C_CHUNK, C_CHUNK), 1)
    tril = (r > c).astype(jnp.bfloat16)

    def chunk(i, offset):
        blk = occ_ref[pl.ds(i * C_CHUNK, C_CHUNK), :]
        ranks_ref[pl.ds(i * C_CHUNK, C_CHUNK), :] = (
            jnp.dot(tril, blk.astype(jnp.bfloat16),
                    preferred_element_type=jnp.float32) + offset)
        return offset + jnp.sum(blk, axis=0, keepdims=True)

    counts = lax.fori_loop(0, T_TOKENS // C_CHUNK, chunk,
                           jnp.zeros((1, N_EXPERTS), jnp.float32))

    cb = jnp.ceil(counts * (1.0 / BLK)) * float(BLK)
    r8 = lax.broadcasted_iota(jnp.int32, (N_EXPERTS, N_EXPERTS), 0)
    c8 = lax.broadcasted_iota(jnp.int32, (N_EXPERTS, N_EXPERTS), 1)
    upper = (r8 < c8).astype(jnp.float32)
    base = jnp.dot(cb, upper, preferred_element_type=jnp.float32)

    eye8 = (r8 == c8).astype(jnp.float32)
    bb_col = lax.dot_general(eye8, base * (1.0 / BLK),
                             (((1,), (1,)), ((), ())),
                             preferred_element_type=jnp.float32)
    blocks = lax.broadcasted_iota(jnp.int32, (1, N_BLOCKS), 1).astype(jnp.float32)
    b2e = jnp.sum((bb_col <= blocks).astype(jnp.int32), axis=0,
                  keepdims=True) - 1
    b2e_ref[...] = b2e

    ranks = ranks_ref[...]
    rank1 = jnp.sum(jnp.where(sel1, ranks, 0.0), axis=1, keepdims=True)
    rank2 = jnp.sum(jnp.where(sel2, ranks, 0.0), axis=1, keepdims=True)
    base1 = jnp.sum(jnp.where(sel1, base, 0.0), axis=1, keepdims=True)
    base2 = jnp.sum(jnp.where(sel2, base, 0.0), axis=1, keepdims=True)
    slot1 = (base1 + rank1).astype(jnp.int32)
    slot2 = (base2 + rank2).astype(jnp.int32)
    slots_ref[...] = jnp.concatenate([slot1, slot2], axis=1)
    wpair_ref[...] = jnp.concatenate([m1 / s, m2 / s], axis=1)


def _route(x, gate_w):
    return pl.pallas_call(
        _route_body,
        in_specs=[
            pl.BlockSpec((T_TOKENS, D_MODEL), lambda: (0, 0)),
            pl.BlockSpec((D_MODEL, N_EXPERTS), lambda: (0, 0)),
        ],
        out_specs=[
            pl.BlockSpec((T_TOKENS, TOP_K), lambda: (0, 0)),
            pl.BlockSpec((T_TOKENS, TOP_K), lambda: (0, 0)),
            pl.BlockSpec((1, N_BLOCKS), lambda: (0, 0)),
            pl.BlockSpec((T_TOKENS, D_HALF), lambda: (0, 0)),
        ],
        out_shape=[
            jax.ShapeDtypeStruct((T_TOKENS, TOP_K), jnp.int32),
            jax.ShapeDtypeStruct((T_TOKENS, TOP_K), jnp.float32),
            jax.ShapeDtypeStruct((1, N_BLOCKS), jnp.int32),
            jax.ShapeDtypeStruct((T_TOKENS, D_HALF), jnp.int32),
        ],
        scratch_shapes=[
            pltpu.VMEM((T_TOKENS, N_EXPERTS), jnp.float32),
            pltpu.VMEM((T_TOKENS, N_EXPERTS), jnp.float32),
        ],
    )(x, gate_w)


def _make_sc_gather():
    info = plsc.get_sparse_core_info()
    NC, NS = info.num_cores, info.num_subcores
    NW = NC * NS
    stripe = PAD_N // NW
    gchunk = stripe // 2
    ppw = N_PAIRS // NS
    GSUB = 8
    zchunk = PAD_N // NS
    mesh = plsc.VectorSubcoreMesh(core_axis_name="c", subcore_axis_name="s")

    tpw = T_TOKENS // NW
    ppg = 2 * tpw

    @functools.partial(
        pl.kernel, mesh=mesh,
        out_type=[
            jax.ShapeDtypeStruct((PAD_N, D_HALF), jnp.int32),
            jax.ShapeDtypeStruct((PAD_N,), jnp.float32),
        ],
        scratch_types=[
            pltpu.VMEM((2, 128), jnp.int32),
            pltpu.VMEM((2, 128), jnp.float32),
            pltpu.VMEM((zchunk,), jnp.float32),
            pltpu.VMEM((stripe,), jnp.float32),
            pltpu.VMEM((ppg,), jnp.int32),
            pltpu.VMEM((tpw, D_HALF), jnp.int32),
            pltpu.VMEM_SHARED((PAD_N,), jnp.float32),
            pltpu.SemaphoreType.DMA,
            pltpu.SemaphoreType.DMA,
        ],
        compiler_params=pltpu.CompilerParams(needs_layout_passes=False),
    )
    def sc_gather(x_hbm, slots_hbm, w_hbm, xs_hbm, ws_hbm,
                  idx_v, wv_v, zerof_v, wstr_v, slot_v, xrows_v,
                  wm_sh, semr, semw):
        cid = lax.axis_index("c")
        sid = lax.axis_index("s")
        wid = sid * NC + cid
        pbase_sc = sid * ppw
        gbase = wid * ppg

        rows_cp = pltpu.async_copy(x_hbm.at[pl.ds(wid * tpw, tpw)],
                                   xrows_v, semr)
        pltpu.sync_copy(slots_hbm.at[pl.ds(gbase, ppg)], slot_v)

        for h in range(2):
            pltpu.sync_copy(slots_hbm.at[pl.ds(pbase_sc + h * 128, 128)],
                            idx_v.at[h])
            pltpu.sync_copy(w_hbm.at[pl.ds(pbase_sc + h * 128, 128)],
                            wv_v.at[h])
        z16f = jnp.zeros((16,), jnp.float32)
        for i in range(zchunk // 16):
            zerof_v[pl.ds(i * 16, 16)] = z16f
        pltpu.sync_copy(zerof_v, wm_sh.at[pl.ds(sid * zchunk, zchunk)])
        plsc.subcore_barrier()
        for h in range(2):
            pltpu.sync_copy(wv_v.at[h], wm_sh.at[idx_v.at[h]])
        plsc.subcore_barrier()
        base = wid * stripe
        pltpu.sync_copy(wm_sh.at[pl.ds(base, stripe)], wstr_v)
        pltpu.sync_copy(wstr_v, ws_hbm.at[pl.ds(base, stripe)])

        rows_cp.wait()
        iota16 = lax.iota(jnp.int32, 16)

        def fire(p, _):
            chunk = slot_v[pl.ds((p >> 4) * 16, 16)]
            s = jnp.sum(jnp.where(iota16 == (p & 15), chunk, 0))
            pltpu.async_copy(xrows_v.at[pl.ds(p >> 1, 1)],
                             xs_hbm.at[pl.ds(s, 1)], semw)
            return 0

        lax.fori_loop(0, ppg, fire, 0)
        for h in range(2):
            pltpu.make_async_copy(x_hbm.at[pl.ds(0, tpw)], xrows_v,
                                  semw).wait()

    return sc_gather


def _gemm_body(b2e_ref, xs_ref, gup_ref, down_ref, w_ref, y_ref):
    xw = xs_ref[...]
    xlo = lax.bitcast_convert_type((xw & 0xFFFF).astype(jnp.int16),
                                   jnp.bfloat16)
    xhi = lax.bitcast_convert_type((xw >> 16).astype(jnp.int16),
                                   jnp.bfloat16)
    xs = jnp.concatenate([xlo, xhi], axis=1)
    gu = jnp.dot(xs, gup_ref[0].astype(jnp.bfloat16),
                 preferred_element_type=jnp.float32)
    g = gu[:, :D_FF]
    u = gu[:, D_FF:]
    act = (g * jax.nn.sigmoid(g) * u).astype(jnp.bfloat16)
    y = jnp.dot(act, down_ref[0].astype(jnp.bfloat16),
                preferred_element_type=jnp.float32)
    r = lax.broadcasted_iota(jnp.int32, (BLK, BLK), 0)
    c = lax.broadcasted_iota(jnp.int32, (BLK, BLK), 1)
    eye = (r == c).astype(jnp.float32)
    wcol = lax.dot_general(eye, w_ref[0], (((1,), (1,)), ((), ())),
                           preferred_element_type=jnp.float32)
    yb = (y * wcol).astype(jnp.bfloat16)
    ylo = lax.bitcast_convert_type(yb[:, :D_HALF], jnp.int16).astype(jnp.int32)
    yhi = lax.bitcast_convert_type(yb[:, D_HALF:], jnp.int16).astype(jnp.int32)
    y_ref[...] = (ylo & 0xFFFF) | (yhi << 16)


def _gemm(b2e, xs, gup, down, ws):
    grid_spec = pltpu.PrefetchScalarGridSpec(
        num_scalar_prefetch=1,
        grid=(N_BLOCKS,),
        in_specs=[
            pl.BlockSpec((BLK, D_HALF), lambda b, b2e: (b, 0)),
            pl.BlockSpec((1, D_MODEL, 2 * D_FF),
                         lambda b, b2e: (b2e[0, b], 0, 0)),
            pl.BlockSpec((1, D_FF, D_MODEL),
                         lambda b, b2e: (b2e[0, b], 0, 0)),
            pl.BlockSpec((1, 1, BLK), lambda b, b2e: (b, 0, 0)),
        ],
        out_specs=pl.BlockSpec((BLK, D_HALF), lambda b, b2e: (b, 0)),
    )
    return pl.pallas_call(
        _gemm_body,
        grid_spec=grid_spec,
        out_shape=jax.ShapeDtypeStruct((PAD_N, D_HALF), jnp.int32),
    )(b2e, xs, gup, down, ws.reshape(N_BLOCKS, 1, BLK))


def _make_sc_combine():
    info = plsc.get_sparse_core_info()
    NC, NS = info.num_cores, info.num_subcores
    NW = NC * NS
    tpw = T_TOKENS // NW
    half = tpw // 2
    mesh = plsc.VectorSubcoreMesh(core_axis_name="c", subcore_axis_name="s")

    nchunk = 4
    tpc = tpw // nchunk
    ppc = 2 * tpc

    @functools.partial(
        pl.kernel, mesh=mesh,
        out_type=jax.ShapeDtypeStruct((T_TOKENS, D_HALF), jnp.int32),
        scratch_types=[
            pltpu.VMEM((2 * tpw,), jnp.int32),
            pltpu.VMEM((2, ppc, D_HALF), jnp.int32),
            pltpu.VMEM((tpc, D_HALF), jnp.int32),
            pltpu.SemaphoreType.DMA,
            pltpu.SemaphoreType.DMA,
        ],
        compiler_params=pltpu.CompilerParams(needs_layout_passes=False),
    )
    def sc_combine(y_hbm, slots_hbm, out_hbm, idx_v, rows_v, out_v,
                   sem0, sem1):
        wid = lax.axis_index("s") * NC + lax.axis_index("c")
        pltpu.sync_copy(slots_hbm.at[pl.ds(wid * 2 * tpw, 2 * tpw)], idx_v)
        nvec = D_MODEL // 16
        sems = (sem0, sem1)
        iota16 = lax.iota(jnp.int32, 16)

        def fire(c, buf):
            def body(i, _):
                p = c * ppc + i
                chunk = idx_v[pl.ds((p >> 4) * 16, 16)]
                s = jnp.sum(jnp.where(iota16 == (p & 15), chunk, 0))
                pltpu.async_copy(y_hbm.at[pl.ds(s, 1)],
                                 rows_v.at[buf, pl.ds(i, 1)], sems[buf])
                return 0
            lax.fori_loop(0, ppc, body, 0)

        def drain(buf):
            pltpu.make_async_copy(y_hbm.at[pl.ds(0, ppc)],
                                  rows_v.at[buf], sems[buf]).wait()

        fire(0, 0)
        for c in range(nchunk):
            buf = c % 2
            if c + 1 < nchunk:
                fire(c + 1, 1 - buf)
            drain(buf)

            def tok(j, _):
                for v in range(D_HALF // 16):
                    a = plsc.bitcast(rows_v[buf, 2 * j, pl.ds(v * 16, 16)],
                                     jnp.bfloat16)
                    b = plsc.bitcast(rows_v[buf, 2 * j + 1,
                                            pl.ds(v * 16, 16)],
                                     jnp.bfloat16)
                    out_v[j, pl.ds(v * 16, 16)] = plsc.bitcast(a + b,
                                                               jnp.int32)
                return 0

            lax.fori_loop(0, tpc, tok, 0)
            pltpu.sync_copy(out_v,
                            out_hbm.at[pl.ds(wid * tpw + c * tpc, tpc)])

    return sc_combine


def kernel(hidden_states, gate_w, gate_up_proj, down_proj):
    batch, seq, d = hidden_states.shape
    x = hidden_states.reshape(batch * seq, d)
    slots2, wpair, b2e, xbf = _route(x, gate_w)
    slots_flat = slots2.reshape(N_PAIRS)
    w_flat = wpair.reshape(N_PAIRS)
    xs, ws = _make_sc_gather()(xbf, slots_flat, w_flat)
    y = _gemm(b2e, xs, gate_up_proj, down_proj, ws)
    out32 = _make_sc_combine()(y, slots_flat)
    pairs = lax.bitcast_convert_type(out32, jnp.bfloat16)
    out = jnp.concatenate([pairs[:, :, 0], pairs[:, :, 1]], axis=1)
    return out.astype(jnp.float32).reshape(batch, seq, d)


# speedup vs baseline: 1.6274x; 1.0257x over previous
import functools

import jax
import jax.numpy as jnp
from jax import lax
from jax.experimental import pallas as pl
from jax.experimental.pallas import tpu as pltpu
from jax.experimental.pallas import tpu_sc as plsc

D_MODEL = 768
N_EXPERTS = 8
TOP_K = 2
D_FF = 768
T_TOKENS = 2048
N_PAIRS = T_TOKENS * TOP_K
BLK = 128
N_BLOCKS = (N_PAIRS + N_EXPERTS * (BLK - 1) + BLK - 1) // BLK
PAD_N = N_BLOCKS * BLK
C_CHUNK = 128
D_HALF = D_MODEL // 2


def _route_body(x_ref, gate_w_ref, slots_ref, wpair_ref, b2e_ref, xbf_ref,
                occ_ref, ranks_ref):
    x = x_ref[...]
    xb = x.astype(jnp.bfloat16)
    lo32 = lax.bitcast_convert_type(xb[:, :D_HALF], jnp.int16).astype(jnp.int32)
    hi32 = lax.bitcast_convert_type(xb[:, D_HALF:], jnp.int16).astype(jnp.int32)
    xbf_ref[...] = (lo32 & 0xFFFF) | (hi32 << 16)
    logits = jnp.dot(x, gate_w_ref[...], preferred_element_type=jnp.float32)
    z = logits - jnp.max(logits, axis=1, keepdims=True)
    ez = jnp.exp(z)
    p = ez / jnp.sum(ez, axis=1, keepdims=True)

    lane = lax.broadcasted_iota(jnp.int32, (T_TOKENS, N_EXPERTS), 1)
    m1 = jnp.max(p, axis=1, keepdims=True)
    i1 = jnp.min(jnp.where(p == m1, lane, N_EXPERTS), axis=1, keepdims=True)
    sel1 = lane == i1
    p2 = jnp.where(sel1, -1.0, p)
    m2 = jnp.max(p2, axis=1, keepdims=True)
    i2 = jnp.min(jnp.where(p2 == m2, lane, N_EXPERTS), axis=1, keepdims=True)
    sel2 = lane == i2
    s = m1 + m2

    occ_ref[...] = jnp.where(sel1 | sel2, 1.0, 0.0)

    r = lax.broadcasted_iota(jnp.int32, (C_CHUNK, C_CHUNK), 0)
    c = lax.broadcasted_iota(jnp.int32, (C_CHUNK, C_CHUNK), 1)
    tril = (r > c).astype(jnp.bfloat16)

    def chunk(i, offset):
        blk = occ_ref[pl.ds(i * C_CHUNK, C_CHUNK), :]
        ranks_ref[pl.ds(i * C_CHUNK, C_CHUNK), :] = (
            jnp.dot(tril, blk.astype(jnp.bfloat16),
                    preferred_element_type=jnp.float32) + offset)
        return offset + jnp.sum(blk, axis=0, keepdims=True)

    counts = lax.fori_loop(0, T_TOKENS // C_CHUNK, chunk,
                           jnp.zeros((1, N_EXPERTS), jnp.float32))

    cb = jnp.ceil(counts * (1.0 / BLK)) * float(BLK)
    r8 = lax.broadcasted_iota(jnp.int32, (N_EXPERTS, N_EXPERTS), 0)
    c8 = lax.broadcasted_iota(jnp.int32, (N_EXPERTS, N_EXPERTS), 1)
    upper = (r8 < c8).astype(jnp.float32)
    base = jnp.dot(cb, upper, preferred_element_type=jnp.float32)

    eye8 = (r8 == c8).astype(jnp.float32)
    bb_col = lax.dot_general(eye8, base * (1.0 / BLK),
                             (((1,), (1,)), ((), ())),
                             preferred_element_type=jnp.float32)
    blocks = lax.broadcasted_iota(jnp.int32, (1, N_BLOCKS), 1).astype(jnp.float32)
    b2e = jnp.sum((bb_col <= blocks).astype(jnp.int32), axis=0,
                  keepdims=True) - 1
    b2e_ref[...] = b2e

    ranks = ranks_ref[...]
    rank1 = jnp.sum(jnp.where(sel1, ranks, 0.0), axis=1, keepdims=True)
    rank2 = jnp.sum(jnp.where(sel2, ranks, 0.0), axis=1, keepdims=True)
    base1 = jnp.sum(jnp.where(sel1, base, 0.0), axis=1, keepdims=True)
    base2 = jnp.sum(jnp.where(sel2, base, 0.0), axis=1, keepdims=True)
    slot1 = (base1 + rank1).astype(jnp.int32)
    slot2 = (base2 + rank2).astype(jnp.int32)
    slots_ref[...] = jnp.concatenate([slot1, slot2], axis=1)
    wpair_ref[...] = jnp.concatenate([m1 / s, m2 / s], axis=1)


def _route(x, gate_w):
    return pl.pallas_call(
        _route_body,
        in_specs=[
            pl.BlockSpec((T_TOKENS, D_MODEL), lambda: (0, 0)),
            pl.BlockSpec((D_MODEL, N_EXPERTS), lambda: (0, 0)),
        ],
        out_specs=[
            pl.BlockSpec((T_TOKENS, TOP_K), lambda: (0, 0)),
            pl.BlockSpec((T_TOKENS, TOP_K), lambda: (0, 0)),
            pl.BlockSpec((1, N_BLOCKS), lambda: (0, 0)),
            pl.BlockSpec((T_TOKENS, D_HALF), lambda: (0, 0)),
        ],
        out_shape=[
            jax.ShapeDtypeStruct((T_TOKENS, TOP_K), jnp.int32),
            jax.ShapeDtypeStruct((T_TOKENS, TOP_K), jnp.float32),
            jax.ShapeDtypeStruct((1, N_BLOCKS), jnp.int32),
            jax.ShapeDtypeStruct((T_TOKENS, D_HALF), jnp.int32),
        ],
        scratch_shapes=[
            pltpu.VMEM((T_TOKENS, N_EXPERTS), jnp.float32),
            pltpu.VMEM((T_TOKENS, N_EXPERTS), jnp.float32),
        ],
    )(x, gate_w)


def _make_sc_gather():
    info = plsc.get_sparse_core_info()
    NC, NS = info.num_cores, info.num_subcores
    NW = NC * NS
    stripe = PAD_N // NW
    gchunk = stripe // 2
    ppw = N_PAIRS // NS
    GSUB = 8
    zchunk = PAD_N // NS
    mesh = plsc.VectorSubcoreMesh(core_axis_name="c", subcore_axis_name="s")

    tpw = T_TOKENS // NW
    ppg = 2 * tpw

    @functools.partial(
        pl.kernel, mesh=mesh,
        out_type=[
            jax.ShapeDtypeStruct((PAD_N, D_HALF), jnp.int32),
            jax.ShapeDtypeStruct((PAD_N,), jnp.float32),
        ],
        scratch_types=[
            pltpu.VMEM((2, 128), jnp.int32),
            pltpu.VMEM((2, 128), jnp.float32),
            pltpu.VMEM((zchunk,), jnp.float32),
            pltpu.VMEM((stripe,), jnp.float32),
            pltpu.VMEM((ppg,), jnp.int32),
            pltpu.VMEM((tpw, D_HALF), jnp.int32),
            pltpu.VMEM_SHARED((PAD_N,), jnp.float32),
            pltpu.SemaphoreType.DMA,
            pltpu.SemaphoreType.DMA,
        ],
        compiler_params=pltpu.CompilerParams(needs_layout_passes=False),
    )
    def sc_gather(x_hbm, slots_hbm, w_hbm, xs_hbm, ws_hbm,
                  idx_v, wv_v, zerof_v, wstr_v, slot_v, xrows_v,
                  wm_sh, semr, semw):
        cid = lax.axis_index("c")
        sid = lax.axis_index("s")
        wid = sid * NC + cid
        pbase_sc = sid * ppw
        gbase = wid * ppg

        rows_cp = pltpu.async_copy(x_hbm.at[pl.ds(wid * tpw, tpw)],
                                   xrows_v, semr)
        pltpu.sync_copy(slots_hbm.at[pl.ds(gbase, ppg)], slot_v)

        for h in range(2):
            pltpu.sync_copy(slots_hbm.at[pl.ds(pbase_sc + h * 128, 128)],
                            idx_v.at[h])
            pltpu.sync_copy(w_hbm.at[pl.ds(pbase_sc + h * 128, 128)],
                            wv_v.at[h])
        z16f = jnp.zeros((16,), jnp.float32)
        for i in range(zchunk // 16):
            zerof_v[pl.ds(i * 16, 16)] = z16f
        pltpu.sync_copy(zerof_v, wm_sh.at[pl.ds(sid * zchunk, zchunk)])
        plsc.subcore_barrier()
        for h in range(2):
            pltpu.sync_copy(wv_v.at[h], wm_sh.at[idx_v.at[h]])
        plsc.subcore_barrier()
        base = wid * stripe
        pltpu.sync_copy(wm_sh.at[pl.ds(base, stripe)], wstr_v)
        pltpu.sync_copy(wstr_v, ws_hbm.at[pl.ds(base, stripe)])

        rows_cp.wait()
        iota16 = lax.iota(jnp.int32, 16)

        def fire(p, _):
            chunk = slot_v[pl.ds((p >> 4) * 16, 16)]
            s = jnp.sum(jnp.where(iota16 == (p & 15), chunk, 0))
            pltpu.async_copy(xrows_v.at[pl.ds(p >> 1, 1)],
                             xs_hbm.at[pl.ds(s, 1)], semw)
            return 0

        lax.fori_loop(0, ppg, fire, 0)
        for h in range(2):
            pltpu.make_async_copy(x_hbm.at[pl.ds(0, tpw)], xrows_v,
                                  semw).wait()

    return sc_gather


def _gemm_body(b2e_ref, xs_ref, gup_ref, down_ref, w_ref, y_ref):
    xw = xs_ref[...]
    xlo = lax.bitcast_convert_type((xw & 0xFFFF).astype(jnp.int16),
                                   jnp.bfloat16)
    xhi = lax.bitcast_convert_type((xw >> 16).astype(jnp.int16),
                                   jnp.bfloat16)
    xs = jnp.concatenate([xlo, xhi], axis=1)
    gu = jnp.dot(xs, gup_ref[0].astype(jnp.bfloat16),
                 preferred_element_type=jnp.float32)
    g = gu[:, :D_FF]
    u = gu[:, D_FF:]
    act = (g * jax.nn.sigmoid(g) * u).astype(jnp.bfloat16)
    y = jnp.dot(act, down_ref[0].astype(jnp.bfloat16),
                preferred_element_type=jnp.float32)
    r = lax.broadcasted_iota(jnp.int32, (BLK, BLK), 0)
    c = lax.broadcasted_iota(jnp.int32, (BLK, BLK), 1)
    eye = (r == c).astype(jnp.float32)
    wcol = lax.dot_general(eye, w_ref[0], (((1,), (1,)), ((), ())),
                           preferred_element_type=jnp.float32)
    yb = (y * wcol).astype(jnp.bfloat16)
    ylo = lax.bitcast_convert_type(yb[:, :D_HALF], jnp.int16).astype(jnp.int32)
    yhi = lax.bitcast_convert_type(yb[:, D_HALF:], jnp.int16).astype(jnp.int32)
    y_ref[...] = (ylo & 0xFFFF) | (yhi << 16)


def _gemm(b2e, xs, gup, down, ws):
    grid_spec = pltpu.PrefetchScalarGridSpec(
        num_scalar_prefetch=1,
        grid=(N_BLOCKS,),
        in_specs=[
            pl.BlockSpec((BLK, D_HALF), lambda b, b2e: (b, 0)),
            pl.BlockSpec((1, D_MODEL, 2 * D_FF),
                         lambda b, b2e: (b2e[0, b], 0, 0)),
            pl.BlockSpec((1, D_FF, D_MODEL),
                         lambda b, b2e: (b2e[0, b], 0, 0)),
            pl.BlockSpec((1, 1, BLK), lambda b, b2e: (b, 0, 0)),
        ],
        out_specs=pl.BlockSpec((BLK, D_HALF), lambda b, b2e: (b, 0)),
    )
    return pl.pallas_call(
        _gemm_body,
        grid_spec=grid_spec,
        out_shape=jax.ShapeDtypeStruct((PAD_N, D_HALF), jnp.int32),
    )(b2e, xs, gup, down, ws.reshape(N_BLOCKS, 1, BLK))


def _make_sc_combine():
    info = plsc.get_sparse_core_info()
    NC, NS = info.num_cores, info.num_subcores
    NW = NC * NS
    tpw = T_TOKENS // NW
    half = tpw // 2
    mesh = plsc.VectorSubcoreMesh(core_axis_name="c", subcore_axis_name="s")

    nchunk = 4
    tpc = tpw // nchunk
    ppc = 2 * tpc

    @functools.partial(
        pl.kernel, mesh=mesh,
        out_type=jax.ShapeDtypeStruct((T_TOKENS, D_MODEL), jnp.float32),
        scratch_types=[
            pltpu.VMEM((2 * tpw,), jnp.int32),
            pltpu.VMEM((2, ppc, D_HALF), jnp.int32),
            pltpu.VMEM((tpc, D_MODEL), jnp.float32),
            pltpu.SemaphoreType.DMA,
            pltpu.SemaphoreType.DMA,
        ],
        compiler_params=pltpu.CompilerParams(needs_layout_passes=False),
    )
    def sc_combine(y_hbm, slots_hbm, out_hbm, idx_v, rows_v, out_v,
                   sem0, sem1):
        wid = lax.axis_index("s") * NC + lax.axis_index("c")
        pltpu.sync_copy(slots_hbm.at[pl.ds(wid * 2 * tpw, 2 * tpw)], idx_v)
        nvec = D_MODEL // 16
        sems = (sem0, sem1)
        iota16 = lax.iota(jnp.int32, 16)

        def fire(c, buf):
            def body(i, _):
                p = c * ppc + i
                chunk = idx_v[pl.ds((p >> 4) * 16, 16)]
                s = jnp.sum(jnp.where(iota16 == (p & 15), chunk, 0))
                pltpu.async_copy(y_hbm.at[pl.ds(s, 1)],
                                 rows_v.at[buf, pl.ds(i, 1)], sems[buf])
                return 0
            lax.fori_loop(0, ppc, body, 0)

        def drain(buf):
            pltpu.make_async_copy(y_hbm.at[pl.ds(0, ppc)],
                                  rows_v.at[buf], sems[buf]).wait()

        fire(0, 0)
        for c in range(nchunk):
            buf = c % 2
            if c + 1 < nchunk:
                fire(c + 1, 1 - buf)
            drain(buf)

            def tok(j, _):
                for v in range(D_HALF // 16):
                    a = plsc.bitcast(rows_v[buf, 2 * j, pl.ds(v * 16, 16)],
                                     jnp.bfloat16)
                    b = plsc.bitcast(rows_v[buf, 2 * j + 1,
                                            pl.ds(v * 16, 16)],
                                     jnp.bfloat16)
                    lo, hi = plsc.unpack(a + b,
                                         format=plsc.PackFormat.INTERLEAVED)
                    out_v[j, pl.ds(v * 16, 16)] = lo
                    out_v[j, pl.ds(D_HALF + v * 16, 16)] = hi
                return 0

            lax.fori_loop(0, tpc, tok, 0)
            pltpu.sync_copy(out_v,
                            out_hbm.at[pl.ds(wid * tpw + c * tpc, tpc)])

    return sc_combine


def kernel(hidden_states, gate_w, gate_up_proj, down_proj):
    batch, seq, d = hidden_states.shape
    x = hidden_states.reshape(batch * seq, d)
    slots2, wpair, b2e, xbf = _route(x, gate_w)
    slots_flat = slots2.reshape(N_PAIRS)
    w_flat = wpair.reshape(N_PAIRS)
    xs, ws = _make_sc_gather()(xbf, slots_flat, w_flat)
    y = _gemm(b2e, xs, gate_up_proj, down_proj, ws)
    out = _make_sc_combine()(y, slots_flat)
    return out.reshape(batch, seq, d)
